# baseline (speedup 1.0000x reference)
import functools

import jax
import jax.numpy as jnp
from jax import lax
from jax.experimental import pallas as pl
from jax.experimental.pallas import tpu as pltpu
from jax.experimental.pallas import tpu_sc as plsc

B = 4
N = 4096
L = 16
NC = 2
NS = 16
NW = NC * NS
WPB = NW // B
RPW = N // WPB
GPB = 4
RBLK = GPB * L
NBLK = RPW // RBLK
NCH = N // L


def _bf16_round(v):
    u = lax.bitcast_convert_type(v, jnp.uint32)
    u = (u + jnp.uint32(0x7FFF) + ((u >> jnp.uint32(16)) & jnp.uint32(1)))
    u = u & jnp.uint32(0xFFFF0000)
    return lax.bitcast_convert_type(u, jnp.float32)


def _nn_body(xyz1f, xyz2f, d1, i1, d2, i2,
             cbuf, rbuf, nA, nB, dbuf, ibuf):
    wid = lax.axis_index("c") * NS + lax.axis_index("s")
    b = wid // WPB
    r0 = (wid % WPB) * RPW

    for src, nbuf in ((xyz1f, nA), (xyz2f, nB)):
        pltpu.sync_copy(src.at[b], cbuf)

        def norm_chunk(ch, carry, nbuf=nbuf):
            s = pl.ds(ch * L, L)
            x = cbuf[0, s]
            y = cbuf[1, s]
            z = cbuf[2, s]
            nbuf[s] = (x * x + y * y) + z * z
            return carry
        lax.fori_loop(0, NCH, norm_chunk, 0)

    for pas in range(2):
        rows_src = xyz1f if pas == 0 else xyz2f
        cand_src = xyz2f if pas == 0 else xyz1f
        nrow = nA if pas == 0 else nB
        ncand = nB if pas == 0 else nA
        d_out = d1 if pas == 0 else d2
        i_out = i1 if pas == 0 else i2

        pltpu.sync_copy(cand_src.at[b], cbuf)
        pltpu.sync_copy(rows_src.at[b, :, pl.ds(r0, RPW)], rbuf)

        def dbl_chunk(ch, carry):
            s = pl.ds(ch * L, L)
            for p in range(3):
                v = _bf16_round(cbuf[p, s])
                cbuf[p, s] = v + v
            return carry
        lax.fori_loop(0, NCH, dbl_chunk, 0)

        def row_round_chunk(ch, carry):
            s = pl.ds(ch * L, L)
            for p in range(3):
                rbuf[p, s] = _bf16_round(rbuf[p, s])
            return carry
        lax.fori_loop(0, RPW // L, row_round_chunk, 0)

        def block(blk, carry):
            base = blk * RBLK
            xs, ys, zs, n1s = [], [], [], []
            for g in range(GPB):
                s = pl.ds(base + g * L, L)
                xs.append(rbuf[0, s])
                ys.append(rbuf[1, s])
                zs.append(rbuf[2, s])
                n1s.append(nrow[pl.ds(r0 + base + g * L, L)])

            best0 = tuple(jnp.full((L,), jnp.inf, jnp.float32)
                          for _ in range(GPB))
            idx0 = tuple(jnp.zeros((L,), jnp.float32) for _ in range(GPB))
            jf0 = jnp.zeros((L,), jnp.float32)

            def inner(ch, c):
                best, idxf, jf = c
                cb = ch * L
                cxv = cbuf[0, pl.ds(cb, L)]
                cyv = cbuf[1, pl.ds(cb, L)]
                czv = cbuf[2, pl.ds(cb, L)]
                cnv = ncand[pl.ds(cb, L)]
                best = list(best)
                idxf = list(idxf)
                for cdx in range(L):
                    cx = jnp.full((L,), cxv[cdx], jnp.float32)
                    cy = jnp.full((L,), cyv[cdx], jnp.float32)
                    cz = jnp.full((L,), czv[cdx], jnp.float32)
                    cn = jnp.full((L,), cnv[cdx], jnp.float32)
                    for g in range(GPB):
                        cr2 = xs[g] * cx + ys[g] * cy + zs[g] * cz
                        dsq = (n1s[g] + cn) - cr2
                        m = dsq < best[g]
                        best[g] = jnp.where(m, dsq, best[g])
                        idxf[g] = jnp.where(m, jf, idxf[g])
                    jf = jf + 1.0
                return (tuple(best), tuple(idxf), jf)

            best, idxf, _ = lax.fori_loop(0, NCH, inner, (best0, idx0, jf0))
            for g in range(GPB):
                s = pl.ds(base + g * L, L)
                dbuf[s] = best[g]
                ibuf[s] = idxf[g].astype(jnp.int32)
            return carry
        lax.fori_loop(0, NBLK, block, 0)

        pltpu.sync_copy(dbuf, d_out.at[b, pl.ds(r0, RPW)])
        pltpu.sync_copy(ibuf, i_out.at[b, pl.ds(r0, RPW)])


@jax.jit
def kernel(xyz1, xyz2):
    mesh = plsc.VectorSubcoreMesh(core_axis_name="c", subcore_axis_name="s")
    f = functools.partial(
        pl.kernel,
        mesh=mesh,
        out_type=[
            jax.ShapeDtypeStruct((B, N), jnp.float32),
            jax.ShapeDtypeStruct((B, N), jnp.int32),
            jax.ShapeDtypeStruct((B, N), jnp.float32),
            jax.ShapeDtypeStruct((B, N), jnp.int32),
        ],
        scratch_types=[
            pltpu.VMEM((3, N), jnp.float32),
            pltpu.VMEM((3, RPW), jnp.float32),
            pltpu.VMEM((N,), jnp.float32),
            pltpu.VMEM((N,), jnp.float32),
            pltpu.VMEM((RPW,), jnp.float32),
            pltpu.VMEM((RPW,), jnp.int32),
        ],
    )(_nn_body)
    x1t = jnp.transpose(xyz1, (0, 2, 1))
    x2t = jnp.transpose(xyz2, (0, 2, 1))
    d1, i1, d2, i2 = f(x1t, x2t)
    return (d1, i1, d2, i2)


# speedup vs baseline: 5.1801x; 5.1801x over previous
import functools

import jax
import jax.numpy as jnp
from jax import lax
from jax.experimental import pallas as pl
from jax.experimental.pallas import tpu as pltpu
from jax.experimental.pallas import tpu_sc as plsc

B = 4
N = 4096
L = 16
NC = 2
NS = 16
NW = NC * NS
WPB = NW // B
RPW = N // WPB
GR = 4
NSB = RPW // L
NCH = N // L
BIG = 1 << 30


def _bf16_round(v):
    u = lax.bitcast_convert_type(v, jnp.uint32)
    u = (u + jnp.uint32(0x7FFF) + ((u >> jnp.uint32(16)) & jnp.uint32(1)))
    u = u & jnp.uint32(0xFFFF0000)
    return lax.bitcast_convert_type(u, jnp.float32)


def _nn_body(xyz1f, xyz2f, d1, i1, d2, i2,
             cbuf, rbuf, nA, nB, dbuf, ibuf):
    wid = lax.axis_index("c") * NS + lax.axis_index("s")
    b = wid // WPB
    r0 = (wid % WPB) * RPW

    for src, nbuf in ((xyz1f, nA), (xyz2f, nB)):
        pltpu.sync_copy(src.at[b], cbuf)

        def norm_chunk(ch, carry, nbuf=nbuf):
            s = pl.ds(ch * L, L)
            x = cbuf[0, s]
            y = cbuf[1, s]
            z = cbuf[2, s]
            nbuf[s] = (x * x + y * y) + z * z
            return carry
        lax.fori_loop(0, NCH, norm_chunk, 0)

    for pas in range(2):
        rows_src = xyz1f if pas == 0 else xyz2f
        cand_src = xyz2f if pas == 0 else xyz1f
        nrow = nA if pas == 0 else nB
        ncand = nB if pas == 0 else nA
        d_out = d1 if pas == 0 else d2
        i_out = i1 if pas == 0 else i2

        pltpu.sync_copy(cand_src.at[b], cbuf)
        pltpu.sync_copy(rows_src.at[b, :, pl.ds(r0, RPW)], rbuf)

        def dbl_chunk(ch, carry):
            s = pl.ds(ch * L, L)
            for p in range(3):
                v = _bf16_round(cbuf[p, s])
                cbuf[p, s] = v + v
            return carry
        lax.fori_loop(0, NCH, dbl_chunk, 0)

        def row_round_chunk(ch, carry):
            s = pl.ds(ch * L, L)
            for p in range(3):
                rbuf[p, s] = _bf16_round(rbuf[p, s])
            return carry
        lax.fori_loop(0, RPW // L, row_round_chunk, 0)

        iotai = lax.iota(jnp.int32, L)
        iotaf = lax.bitcast_convert_type(
            jnp.bitwise_or(iotai, jnp.int32(0x4B000000)),
            jnp.float32) - 8388608.0

        def superblock(sb, carry):
            base = sb * L
            xs = rbuf[0, pl.ds(base, L)]
            ys = rbuf[1, pl.ds(base, L)]
            zs = rbuf[2, pl.ds(base, L)]
            ns = nrow[pl.ds(r0 + base, L)]

            dacc = jnp.zeros((L,), jnp.float32)
            iacc = jnp.zeros((L,), jnp.float32)
            for g in range(L // GR):
                xr, yr, zr, nr = [], [], [], []
                for r in range(GR):
                    lane = g * GR + r
                    xr.append(jnp.full((L,), xs[lane], jnp.float32))
                    yr.append(jnp.full((L,), ys[lane], jnp.float32))
                    zr.append(jnp.full((L,), zs[lane], jnp.float32))
                    nr.append(jnp.full((L,), ns[lane], jnp.float32))

                best0 = tuple(jnp.full((L,), jnp.inf, jnp.float32)
                              for _ in range(GR))
                idx0 = tuple(jnp.zeros((L,), jnp.float32)
                             for _ in range(GR))

                def inner(ch, c):
                    best, idxf, jfv = c
                    cb = ch * L
                    cxv = cbuf[0, pl.ds(cb, L)]
                    cyv = cbuf[1, pl.ds(cb, L)]
                    czv = cbuf[2, pl.ds(cb, L)]
                    cnv = ncand[pl.ds(cb, L)]
                    best = list(best)
                    idxf = list(idxf)
                    for r in range(GR):
                        cr2 = xr[r] * cxv + yr[r] * cyv + zr[r] * czv
                        dsq = (nr[r] + cnv) - cr2
                        m = dsq < best[r]
                        best[r] = jnp.where(m, dsq, best[r])
                        idxf[r] = jnp.where(m, jfv, idxf[r])
                    return (tuple(best), tuple(idxf), jfv + 16.0)

                best, idxf, _ = lax.fori_loop(
                    0, NCH, inner, (best0, idx0, iotaf))

                for r in range(GR):
                    v, ix = best[r], idxf[r]
                    vm = jnp.full((L,), v[0], jnp.float32)
                    im = jnp.full((L,), ix[0], jnp.float32)
                    for i in range(1, L):
                        sv = jnp.full((L,), v[i], jnp.float32)
                        si = jnp.full((L,), ix[i], jnp.float32)
                        c1 = sv < vm
                        c2 = vm < sv
                        ct = si < im
                        im = jnp.where(
                            c1, si, jnp.where(c2, im,
                                              jnp.where(ct, si, im)))
                        vm = jnp.where(c1, sv, vm)
                    cf = float(g * GR + r)
                    dlt = iotaf - cf
                    lanem = dlt * dlt < 0.25
                    dacc = jnp.where(lanem, vm, dacc)
                    iacc = jnp.where(lanem, im, iacc)

            dbuf[pl.ds(base, L)] = dacc
            ibuf[pl.ds(base, L)] = iacc.astype(jnp.int32)
            return carry
        lax.fori_loop(0, NSB, superblock, 0)

        pltpu.sync_copy(dbuf, d_out.at[b, pl.ds(r0, RPW)])
        pltpu.sync_copy(ibuf, i_out.at[b, pl.ds(r0, RPW)])


@jax.jit
def kernel(xyz1, xyz2):
    mesh = plsc.VectorSubcoreMesh(core_axis_name="c", subcore_axis_name="s")
    f = functools.partial(
        pl.kernel,
        mesh=mesh,
        out_type=[
            jax.ShapeDtypeStruct((B, N), jnp.float32),
            jax.ShapeDtypeStruct((B, N), jnp.int32),
            jax.ShapeDtypeStruct((B, N), jnp.float32),
            jax.ShapeDtypeStruct((B, N), jnp.int32),
        ],
        scratch_types=[
            pltpu.VMEM((3, N), jnp.float32),
            pltpu.VMEM((3, RPW), jnp.float32),
            pltpu.VMEM((N,), jnp.float32),
            pltpu.VMEM((N,), jnp.float32),
            pltpu.VMEM((RPW,), jnp.float32),
            pltpu.VMEM((RPW,), jnp.int32),
        ],
    )(_nn_body)
    x1t = jnp.transpose(xyz1, (0, 2, 1))
    x2t = jnp.transpose(xyz2, (0, 2, 1))
    d1, i1, d2, i2 = f(x1t, x2t)
    return (d1, i1, d2, i2)
